# baseline (device time: 72278 ns/iter reference)
import jax
import jax.numpy as jnp
from jax import lax
from jax.experimental import pallas as pl
from jax.experimental.pallas import tpu as pltpu


def kernel(O, Wo):
    B, S, Hs, D = O.shape
    K = Hs * D
    N = Wo.shape[1]
    S_half = S // 2

    O = O.reshape(B, S, K)

    CH = 4
    R = S_half // CH
    NCHUNK = B * CH

    def body(o_ref, wo_ref, out_ref, wo_vmem, o_part, o_own, out_acc,
             send_buf, recv_buf, wo_sem, part_sems, own_sems, out_sem,
             send_sems, recv_sems):
        my_x = lax.axis_index("x")
        my_y = lax.axis_index("y")
        my_z = lax.axis_index("z")
        partner = (my_x, 1 - my_y, my_z)

        part_start = (1 - my_y) * S_half
        my_start = my_y * S_half

        wo_dma = pltpu.make_async_copy(wo_ref, wo_vmem, wo_sem)
        wo_dma.start()
        part_dmas = []
        for b in range(B):
            for c in range(CH):
                idx = b * CH + c
                cp = pltpu.make_async_copy(
                    o_ref.at[b, pl.ds(part_start + c * R, R), :],
                    o_part.at[b, pl.ds(c * R, R), :],
                    part_sems.at[idx],
                )
                cp.start()
                part_dmas.append(cp)
        own_dmas = []
        for b in range(B):
            cp = pltpu.make_async_copy(
                o_ref.at[b, pl.ds(my_start, S_half), :],
                o_own.at[b],
                own_sems.at[b],
            )
            cp.start()
            own_dmas.append(cp)

        barrier_sem = pltpu.get_barrier_semaphore()
        pl.semaphore_signal(
            barrier_sem, inc=1,
            device_id=partner, device_id_type=pl.DeviceIdType.MESH,
        )
        pl.semaphore_wait(barrier_sem, 1)

        wo_dma.wait()
        wo = wo_vmem[...].astype(jnp.bfloat16)

        rdmas = []
        for b in range(B):
            for c in range(CH):
                idx = b * CH + c
                part_dmas[idx].wait()
                o_b = o_part[b, c * R:(c + 1) * R, :].astype(jnp.bfloat16)
                send_buf[b, c * R:(c + 1) * R, :] = jnp.dot(
                    o_b, wo, preferred_element_type=jnp.float32
                ).astype(jnp.bfloat16)
                rdma = pltpu.make_async_remote_copy(
                    src_ref=send_buf.at[b, c * R:(c + 1) * R, :],
                    dst_ref=recv_buf.at[b, c * R:(c + 1) * R, :],
                    send_sem=send_sems.at[idx],
                    recv_sem=recv_sems.at[idx],
                    device_id=partner,
                    device_id_type=pl.DeviceIdType.MESH,
                )
                rdma.start()
                rdmas.append(rdma)

        for b in range(B):
            own_dmas[b].wait()
            o_b = o_own[b, :, :].astype(jnp.bfloat16)
            out_acc[b, :, :] = jnp.dot(
                o_b, wo, preferred_element_type=jnp.float32
            )

        out_copies = []
        for b in range(B):
            for c in range(CH):
                idx = b * CH + c
                rdmas[idx].wait_recv()
                out_acc[b, c * R:(c + 1) * R, :] += recv_buf[
                    b, c * R:(c + 1) * R, :
                ].astype(jnp.float32)
                cp = pltpu.make_async_copy(
                    out_acc.at[b, pl.ds(c * R, R), :],
                    out_ref.at[b, pl.ds(c * R, R), :],
                    out_sem,
                )
                cp.start()
                out_copies.append(cp)
        for cp in out_copies:
            cp.wait()
        for rdma in rdmas:
            rdma.wait_send()

    return pl.pallas_call(
        body,
        out_shape=jax.ShapeDtypeStruct((B, S_half, N), jnp.float32),
        in_specs=[
            pl.BlockSpec(memory_space=pl.ANY),
            pl.BlockSpec(memory_space=pl.ANY),
        ],
        out_specs=pl.BlockSpec(memory_space=pl.ANY),
        scratch_shapes=[
            pltpu.VMEM((K, N), jnp.float32),
            pltpu.VMEM((B, S_half, K), jnp.float32),
            pltpu.VMEM((B, S_half, K), jnp.float32),
            pltpu.VMEM((B, S_half, N), jnp.float32),
            pltpu.VMEM((B, S_half, N), jnp.bfloat16),
            pltpu.VMEM((B, S_half, N), jnp.bfloat16),
            pltpu.SemaphoreType.DMA,
            pltpu.SemaphoreType.DMA((NCHUNK,)),
            pltpu.SemaphoreType.DMA((B,)),
            pltpu.SemaphoreType.DMA,
            pltpu.SemaphoreType.DMA((NCHUNK,)),
            pltpu.SemaphoreType.DMA((NCHUNK,)),
        ],
        compiler_params=pltpu.CompilerParams(
            collective_id=0,
            vmem_limit_bytes=96 * 1024 * 1024,
        ),
    )(O, Wo)
